# initial kernel scaffold (unmeasured)
import jax
import jax.numpy as jnp
from jax import lax
from jax.experimental import pallas as pl
from jax.experimental.pallas import tpu as pltpu

N_DEV = 16


def kernel(x, w_mat):
    m_loc, k = x.shape
    _, n = w_mat.shape
    n_loc = n // N_DEV

    def body(x_ref, w_hbm, out_ref, y_buf, w_buf, maxes,
             w_sems, send_sems, recv_sems, max_send_sems, max_recv_sems):
        my = lax.axis_index("i")

        barrier = pltpu.get_barrier_semaphore()
        for d in range(1, N_DEV):
            peer = (my + d) % N_DEV
            pl.semaphore_signal(barrier, inc=1, device_id=(peer,),
                                device_id_type=pl.DeviceIdType.MESH)
        pl.semaphore_wait(barrier, N_DEV - 1)

        order = list(range(1, N_DEV)) + [0]

        def start_w_copy(idx, slot):
            t = (my + order[idx]) % N_DEV
            pltpu.make_async_copy(
                w_hbm.at[:, pl.ds(t * n_loc, n_loc)],
                w_buf.at[slot],
                w_sems.at[slot],
            ).start()

        def wait_w_copy(slot):
            pltpu.make_async_copy(
                w_hbm.at[:, pl.ds(0, n_loc)],
                w_buf.at[slot],
                w_sems.at[slot],
            ).wait()

        start_w_copy(0, 0)
        m_acc = jnp.float32(0.0)
        for idx, d in enumerate(order):
            slot = idx % 2
            wait_w_copy(slot)
            if idx + 1 < N_DEV:
                start_w_copy(idx + 1, 1 - slot)
            y = jnp.maximum(
                jnp.dot(x_ref[:], w_buf[slot],
                        preferred_element_type=jnp.float32),
                0.0,
            )
            m_acc = jnp.maximum(m_acc, jnp.max(y))
            if d == 0:
                out_ref[pl.ds(my * m_loc, m_loc), :] = y
            else:
                y_buf[d] = y
                t = (my + d) % N_DEV
                pltpu.make_async_remote_copy(
                    src_ref=y_buf.at[d],
                    dst_ref=out_ref.at[pl.ds(my * m_loc, m_loc), :],
                    send_sem=send_sems.at[d],
                    recv_sem=recv_sems.at[d],
                    device_id=(t,),
                    device_id_type=pl.DeviceIdType.MESH,
                ).start()

        maxes[0] = jnp.full((1, 128), m_acc, jnp.float32)
        for d in range(1, N_DEV):
            t = (my + d) % N_DEV
            pltpu.make_async_remote_copy(
                src_ref=maxes.at[0],
                dst_ref=maxes.at[d],
                send_sem=max_send_sems.at[d],
                recv_sem=max_recv_sems.at[d],
                device_id=(t,),
                device_id_type=pl.DeviceIdType.MESH,
            ).start()

        for d in range(1, N_DEV):
            s = (my - d) % N_DEV
            pltpu.make_async_remote_copy(
                src_ref=y_buf.at[d],
                dst_ref=out_ref.at[pl.ds(s * m_loc, m_loc), :],
                send_sem=send_sems.at[d],
                recv_sem=recv_sems.at[d],
                device_id=(my,),
                device_id_type=pl.DeviceIdType.MESH,
            ).wait_recv()
        for d in range(1, N_DEV):
            pltpu.make_async_remote_copy(
                src_ref=maxes.at[0],
                dst_ref=maxes.at[d],
                send_sem=max_send_sems.at[d],
                recv_sem=max_recv_sems.at[d],
                device_id=(my,),
                device_id_type=pl.DeviceIdType.MESH,
            ).wait_recv()
        for d in range(1, N_DEV):
            t = (my + d) % N_DEV
            pltpu.make_async_remote_copy(
                src_ref=y_buf.at[d],
                dst_ref=out_ref.at[pl.ds(my * m_loc, m_loc), :],
                send_sem=send_sems.at[d],
                recv_sem=recv_sems.at[d],
                device_id=(t,),
                device_id_type=pl.DeviceIdType.MESH,
            ).wait_send()
            pltpu.make_async_remote_copy(
                src_ref=maxes.at[0],
                dst_ref=maxes.at[d],
                send_sem=max_send_sems.at[d],
                recv_sem=max_recv_sems.at[d],
                device_id=(t,),
                device_id_type=pl.DeviceIdType.MESH,
            ).wait_send()

        g = jnp.max(maxes[...])
        scale = g / 448.0
        q = (out_ref[...] / scale).astype(jnp.float8_e4m3fn)
        out_ref[...] = q.astype(jnp.float32) * scale

    return pl.pallas_call(
        body,
        out_shape=jax.ShapeDtypeStruct((m_loc * N_DEV, n_loc), jnp.float32),
        in_specs=[
            pl.BlockSpec(memory_space=pltpu.VMEM),
            pl.BlockSpec(memory_space=pltpu.ANY),
        ],
        out_specs=pl.BlockSpec(memory_space=pltpu.VMEM),
        scratch_shapes=[
            pltpu.VMEM((N_DEV, m_loc, n_loc), jnp.float32),
            pltpu.VMEM((2, k, n_loc), jnp.float32),
            pltpu.VMEM((N_DEV, 1, 128), jnp.float32),
            pltpu.SemaphoreType.DMA((2,)),
            pltpu.SemaphoreType.DMA((N_DEV,)),
            pltpu.SemaphoreType.DMA((N_DEV,)),
            pltpu.SemaphoreType.DMA((N_DEV,)),
            pltpu.SemaphoreType.DMA((N_DEV,)),
        ],
        compiler_params=pltpu.CompilerParams(collective_id=0),
    )(x, w_mat)


# baseline (device time: 125813 ns/iter reference)
import jax
import jax.numpy as jnp
from jax import lax
from jax.experimental import pallas as pl
from jax.experimental.pallas import tpu as pltpu

N_DEV = 16


def kernel(x, w_mat):
    m_loc, k = x.shape
    _, n = w_mat.shape
    n_loc = n // N_DEV

    def body(x_ref, w_hbm, out_ref, y_buf, w_buf, maxes,
             w_sems, send_sems, recv_sems, max_send_sems, max_recv_sems):
        my = lax.axis_index("i")

        barrier = pltpu.get_barrier_semaphore()
        for d in range(1, N_DEV):
            peer = (my + d) % N_DEV
            pl.semaphore_signal(barrier, inc=1, device_id=(peer,),
                                device_id_type=pl.DeviceIdType.MESH)
        pl.semaphore_wait(barrier, N_DEV - 1)

        order = list(range(1, N_DEV)) + [0]

        def start_w_copy(idx, slot):
            t = (my + order[idx]) % N_DEV
            pltpu.make_async_copy(
                w_hbm.at[:, pl.ds(t * n_loc, n_loc)],
                w_buf.at[slot],
                w_sems.at[slot],
            ).start()

        def wait_w_copy(slot):
            pltpu.make_async_copy(
                w_hbm.at[:, pl.ds(0, n_loc)],
                w_buf.at[slot],
                w_sems.at[slot],
            ).wait()

        start_w_copy(0, 0)
        m_acc = jnp.float32(0.0)
        for idx, d in enumerate(order):
            slot = idx % 2
            wait_w_copy(slot)
            if idx + 1 < N_DEV:
                start_w_copy(idx + 1, 1 - slot)
            y = jnp.maximum(
                jnp.dot(x_ref[:], w_buf[slot],
                        preferred_element_type=jnp.float32),
                0.0,
            )
            m_acc = jnp.maximum(m_acc, jnp.max(y))
            if d == 0:
                out_ref[pl.ds(my * m_loc, m_loc), :] = y
            else:
                y_buf[d] = y
                t = (my + d) % N_DEV
                pltpu.make_async_remote_copy(
                    src_ref=y_buf.at[d],
                    dst_ref=out_ref.at[pl.ds(my * m_loc, m_loc), :],
                    send_sem=send_sems.at[d],
                    recv_sem=recv_sems.at[d],
                    device_id=(t,),
                    device_id_type=pl.DeviceIdType.MESH,
                ).start()

        maxes[0] = jnp.full((1, 128), m_acc, jnp.float32)
        for d in range(1, N_DEV):
            t = (my + d) % N_DEV
            pltpu.make_async_remote_copy(
                src_ref=maxes.at[0],
                dst_ref=maxes.at[d],
                send_sem=max_send_sems.at[d],
                recv_sem=max_recv_sems.at[d],
                device_id=(t,),
                device_id_type=pl.DeviceIdType.MESH,
            ).start()

        for d in range(1, N_DEV):
            s = (my - d) % N_DEV
            pltpu.make_async_remote_copy(
                src_ref=y_buf.at[d],
                dst_ref=out_ref.at[pl.ds(s * m_loc, m_loc), :],
                send_sem=send_sems.at[d],
                recv_sem=recv_sems.at[d],
                device_id=(my,),
                device_id_type=pl.DeviceIdType.MESH,
            ).wait_recv()
        for d in range(1, N_DEV):
            pltpu.make_async_remote_copy(
                src_ref=maxes.at[0],
                dst_ref=maxes.at[d],
                send_sem=max_send_sems.at[d],
                recv_sem=max_recv_sems.at[d],
                device_id=(my,),
                device_id_type=pl.DeviceIdType.MESH,
            ).wait_recv()
        for d in range(1, N_DEV):
            t = (my + d) % N_DEV
            pltpu.make_async_remote_copy(
                src_ref=y_buf.at[d],
                dst_ref=out_ref.at[pl.ds(my * m_loc, m_loc), :],
                send_sem=send_sems.at[d],
                recv_sem=recv_sems.at[d],
                device_id=(t,),
                device_id_type=pl.DeviceIdType.MESH,
            ).wait_send()
            pltpu.make_async_remote_copy(
                src_ref=maxes.at[0],
                dst_ref=maxes.at[d],
                send_sem=max_send_sems.at[d],
                recv_sem=max_recv_sems.at[d],
                device_id=(t,),
                device_id_type=pl.DeviceIdType.MESH,
            ).wait_send()

        g = jnp.max(maxes[...])
        scale = g / 448.0
        q = (out_ref[...] / scale).astype(jnp.float8_e4m3fn)
        out_ref[...] = q.astype(jnp.float32) * scale

    return pl.pallas_call(
        body,
        out_shape=jax.ShapeDtypeStruct((m_loc * N_DEV, n_loc), jnp.float32),
        in_specs=[
            pl.BlockSpec(memory_space=pltpu.VMEM),
            pl.BlockSpec(memory_space=pl.ANY),
        ],
        out_specs=pl.BlockSpec(memory_space=pltpu.VMEM),
        scratch_shapes=[
            pltpu.VMEM((N_DEV, m_loc, n_loc), jnp.float32),
            pltpu.VMEM((2, k, n_loc), jnp.float32),
            pltpu.VMEM((N_DEV, 1, 128), jnp.float32),
            pltpu.SemaphoreType.DMA((2,)),
            pltpu.SemaphoreType.DMA((N_DEV,)),
            pltpu.SemaphoreType.DMA((N_DEV,)),
            pltpu.SemaphoreType.DMA((N_DEV,)),
            pltpu.SemaphoreType.DMA((N_DEV,)),
        ],
        compiler_params=pltpu.CompilerParams(collective_id=0),
    )(x, w_mat)


# device time: 79535 ns/iter; 1.5819x vs baseline; 1.5819x over previous
import jax
import jax.numpy as jnp
from jax import lax
from jax.experimental import pallas as pl
from jax.experimental.pallas import tpu as pltpu

N_DEV = 16
W_BUFS = 3
W_SUBS = 4


def kernel(x, w_mat):
    m_loc, k = x.shape
    _, n = w_mat.shape
    n_loc = n // N_DEV
    rows = k // W_SUBS
    f8 = jnp.float8_e4m3fn

    def body(x_ref, w_hbm, out_ref, w_buf, q_send, q_recv, maxes,
             w_sems, q_send_sems, q_recv_sems, max_send_sems, max_recv_sems):
        my = lax.axis_index("i")

        barrier = pltpu.get_barrier_semaphore()
        for d in range(1, N_DEV):
            peer = (my + d) % N_DEV
            pl.semaphore_signal(barrier, inc=1, device_id=(peer,),
                                device_id_type=pl.DeviceIdType.MESH)
        pl.semaphore_wait(barrier, N_DEV - 1)

        order = list(range(1, N_DEV)) + [0]

        def start_w(idx, slot):
            t = (my + order[idx]) % N_DEV
            for j in range(W_SUBS):
                pltpu.make_async_copy(
                    w_hbm.at[pl.ds(j * rows, rows), pl.ds(t * n_loc, n_loc)],
                    w_buf.at[slot, pl.ds(j * rows, rows), :],
                    w_sems.at[slot, j]).start()

        def wait_w(slot):
            for j in range(W_SUBS):
                pltpu.make_async_copy(
                    w_hbm.at[pl.ds(0, rows), pl.ds(0, n_loc)],
                    w_buf.at[slot, pl.ds(j * rows, rows), :],
                    w_sems.at[slot, j]).wait()

        for p in range(W_BUFS - 1):
            start_w(p, p)
        m_acc = jnp.float32(0.0)
        for idx, d in enumerate(order):
            slot = idx % W_BUFS
            wait_w(slot)
            nxt = idx + W_BUFS - 1
            if nxt < N_DEV:
                start_w(nxt, nxt % W_BUFS)
            y = jnp.maximum(
                jnp.dot(x_ref[:], w_buf[slot],
                        preferred_element_type=jnp.float32),
                0.0)
            m_acc = jnp.maximum(m_acc, jnp.max(y))
            t = (my + d) % N_DEV
            out_ref[pl.ds(t * m_loc, m_loc), :] = y

        maxes[0] = jnp.full((1, 128), m_acc, jnp.float32)
        for d in range(1, N_DEV):
            t = (my + d) % N_DEV
            pltpu.make_async_remote_copy(
                src_ref=maxes.at[0], dst_ref=maxes.at[d],
                send_sem=max_send_sems.at[d],
                recv_sem=max_recv_sems.at[d],
                device_id=(t,),
                device_id_type=pl.DeviceIdType.MESH).start()
        for d in range(1, N_DEV):
            pltpu.make_async_remote_copy(
                src_ref=maxes.at[0], dst_ref=maxes.at[d],
                send_sem=max_send_sems.at[d],
                recv_sem=max_recv_sems.at[d],
                device_id=(my,),
                device_id_type=pl.DeviceIdType.MESH).wait_recv()

        g = jnp.max(maxes[...])
        inv = 448.0 / g
        scale = g / 448.0

        for d in range(1, N_DEV):
            t = (my + d) % N_DEV
            q_send[d] = (out_ref[pl.ds(t * m_loc, m_loc), :] * inv).astype(f8)
            pltpu.make_async_remote_copy(
                src_ref=q_send.at[d],
                dst_ref=q_recv.at[d],
                send_sem=q_send_sems.at[d],
                recv_sem=q_recv_sems.at[d],
                device_id=(t,),
                device_id_type=pl.DeviceIdType.MESH).start()
        out_ref[pl.ds(my * m_loc, m_loc), :] = (
            (out_ref[pl.ds(my * m_loc, m_loc), :] * inv).astype(f8)
            .astype(jnp.float32) * scale)

        for d in range(1, N_DEV):
            s = (my - d) % N_DEV
            pltpu.make_async_remote_copy(
                src_ref=q_send.at[d],
                dst_ref=q_recv.at[d],
                send_sem=q_send_sems.at[d],
                recv_sem=q_recv_sems.at[d],
                device_id=(my,),
                device_id_type=pl.DeviceIdType.MESH).wait_recv()
            out_ref[pl.ds(s * m_loc, m_loc), :] = (
                q_recv[d].astype(jnp.float32) * scale)

        for d in range(1, N_DEV):
            t = (my + d) % N_DEV
            pltpu.make_async_remote_copy(
                src_ref=q_send.at[d],
                dst_ref=q_recv.at[d],
                send_sem=q_send_sems.at[d],
                recv_sem=q_recv_sems.at[d],
                device_id=(t,),
                device_id_type=pl.DeviceIdType.MESH).wait_send()
            pltpu.make_async_remote_copy(
                src_ref=maxes.at[0], dst_ref=maxes.at[d],
                send_sem=max_send_sems.at[d],
                recv_sem=max_recv_sems.at[d],
                device_id=(t,),
                device_id_type=pl.DeviceIdType.MESH).wait_send()

    return pl.pallas_call(
        body,
        out_shape=jax.ShapeDtypeStruct((m_loc * N_DEV, n_loc), jnp.float32),
        in_specs=[
            pl.BlockSpec(memory_space=pltpu.VMEM),
            pl.BlockSpec(memory_space=pl.ANY),
        ],
        out_specs=pl.BlockSpec(memory_space=pltpu.VMEM),
        scratch_shapes=[
            pltpu.VMEM((W_BUFS, k, n_loc), jnp.float32),
            pltpu.VMEM((N_DEV, m_loc, n_loc), f8),
            pltpu.VMEM((N_DEV, m_loc, n_loc), f8),
            pltpu.VMEM((N_DEV, 1, 128), jnp.float32),
            pltpu.SemaphoreType.DMA((W_BUFS, W_SUBS)),
            pltpu.SemaphoreType.DMA((N_DEV,)),
            pltpu.SemaphoreType.DMA((N_DEV,)),
            pltpu.SemaphoreType.DMA((N_DEV,)),
            pltpu.SemaphoreType.DMA((N_DEV,)),
        ],
        compiler_params=pltpu.CompilerParams(collective_id=0),
    )(x, w_mat)
